# baseline (speedup 1.0000x reference)
import functools

import jax
import jax.numpy as jnp
from jax import lax
from jax.experimental import pallas as pl
from jax.experimental.pallas import tpu as pltpu
from jax.experimental.pallas import tpu_sc as plsc

_L = 4096
_C2 = 64
_BM = 512
_NC = 2
_NS = 16
_NW = _NC * _NS
_CHUNK = _L // _NW
_LOG2E = 1.4426950408889634
_SC_PARAMS = pltpu.CompilerParams(
    needs_layout_passes=False, use_tc_tiling_on_sc=False)


def _sc_compact_gather(mask_hbm, latT_hbm, vT_hbm,
                       qT_hbm, kT_hbm, vkT_hbm, oidx_hbm, counts_hbm,
                       mf_v, qidx_v, kidx_v, oidx_v, idxq_v, idxk_v,
                       rq_v, rk_v, rv_v, cvec_v, sem):
    wid = lax.axis_index("s") * _NC + lax.axis_index("c")
    base = pl.multiple_of(wid * _CHUNK, _CHUNK)
    pltpu.sync_copy(mask_hbm, mf_v)

    lane = lax.iota(jnp.int32, 16)

    def prefill(t, carry):
        off = pl.multiple_of(t * 16, 16)
        kidx_v[pl.ds(off, 16)] = lane + t * 16
        return carry

    lax.fori_loop(0, _L // 16, prefill, jnp.int32(0))

    def body(t, carry):
        cm, ck = carry
        off = pl.multiple_of(t * 16, 16)
        f = mf_v[pl.ds(off, 16)]
        gid = lane + t * 16
        m = f > 0
        cs = plsc.cumsum(f)
        s = jnp.sum(f)
        plsc.store_compressed(qidx_v.at[pl.ds(cm, 16)], gid, mask=m)
        plsc.store_compressed(kidx_v.at[pl.ds(ck, 16)], gid, mask=f < 1)
        oidx_v[pl.ds(off, 16)] = jnp.where(m, cm + cs - 1, _L - 1)
        return cm + s, ck + (16 - s)

    m_cnt, _ = lax.fori_loop(0, _L // 16, body,
                             (jnp.int32(0), jnp.int32(0)))

    cvec_v[...] = jnp.where(lane == 0, m_cnt,
                            jnp.where(lane == 1, _L - m_cnt, 0))

    @pl.when(wid == 0)
    def _write_counts():
        pltpu.sync_copy(cvec_v, counts_hbm)

    pltpu.sync_copy(oidx_v.at[pl.ds(base, _CHUNK)],
                    oidx_hbm.at[pl.ds(base, _CHUNK)])

    for j in range(_CHUNK // 16):
        q16 = qidx_v[pl.ds(base + j * 16, 16)]
        idxq_v[pl.ds(j * 16, 16)] = jnp.minimum(jnp.maximum(q16, 0), _L - 1)
        idxk_v[pl.ds(j * 16, 16)] = kidx_v[pl.ds(base + j * 16, 16)]
    cq = pltpu.async_copy(latT_hbm.at[idxq_v], rq_v, sem)
    ck = pltpu.async_copy(latT_hbm.at[idxk_v], rk_v, sem)
    cv = pltpu.async_copy(vT_hbm.at[idxk_v], rv_v, sem)
    cq.wait()
    pltpu.sync_copy(rq_v, qT_hbm.at[pl.ds(base, _CHUNK)])
    ck.wait()
    pltpu.sync_copy(rk_v, kT_hbm.at[pl.ds(base, _CHUNK)])
    cv.wait()
    pltpu.sync_copy(rv_v, vkT_hbm.at[pl.ds(base, _CHUNK)])


def _sc_out_gather(o_hbm, oidx_hbm, shiftT_hbm, idx_v, rows_v, sem):
    wid = lax.axis_index("s") * _NC + lax.axis_index("c")
    base = pl.multiple_of(wid * _CHUNK, _CHUNK)
    pltpu.sync_copy(oidx_hbm.at[pl.ds(base, _CHUNK)], idx_v)
    pltpu.async_copy(o_hbm.at[idx_v], rows_v, sem).wait()
    pltpu.sync_copy(rows_v, shiftT_hbm.at[pl.ds(base, _CHUNK)])


def _pre_transpose(feat_ref, latT_ref, vT_ref):
    c2 = vT_ref.shape[1]
    latT_ref[...] = jnp.transpose(feat_ref[c2:, :])
    vT_ref[...] = jnp.transpose(feat_ref[:c2, :])


def _attn_block(counts_ref, qT_ref, kT_ref, vkT_ref, out_ref, kn_ref, va_ref):
    i = pl.program_id(0)
    m_cnt = counts_ref[0]
    k_cnt = counts_ref[1]

    @pl.when(i == 0)
    def _prep():
        kt = kT_ref[...]
        norm = jnp.sqrt(jnp.sum(kt * kt, axis=1, keepdims=True)) + 1e-4
        kscale = jnp.where(k_cnt > 0, _LOG2E, 0.0)
        kn_ref[...] = kt * (kscale / norm)
        riota = lax.broadcasted_iota(jnp.int32, (_L, 1), 0)
        kvalid = jnp.where(k_cnt > 0, (riota < k_cnt).astype(jnp.float32), 1.0)
        vkb = (vkT_ref[...] * kvalid).astype(jnp.bfloat16)
        va_ref[...] = jnp.concatenate(
            [vkb, kvalid.astype(jnp.bfloat16),
             jnp.zeros((_L, 63), jnp.bfloat16)], axis=1)

    blk_active = i * _BM < m_cnt

    @pl.when(blk_active)
    def _compute():
        q = qT_ref[...]
        s = lax.dot_general(q, kn_ref[...], (((1,), (1,)), ((), ())),
                            preferred_element_type=jnp.float32)
        e = jnp.exp2(s).astype(jnp.bfloat16)
        oa = lax.dot_general(e, va_ref[...], (((1,), (0,)), ((), ())),
                             preferred_element_type=jnp.float32)
        o = oa[:, :_C2]
        d = oa[:, _C2:_C2 + 1]
        rmask = (i * _BM + lax.broadcasted_iota(jnp.int32, (_BM, 1), 0)) < m_cnt
        out_ref[...] = jnp.where(rmask, o / d, 0.0)

    @pl.when(jnp.logical_not(blk_active))
    def _zero():
        out_ref[...] = jnp.zeros_like(out_ref)


def _post_assemble(feat_ref, shiftT_ref, out_ref):
    c = feat_ref.shape[0]
    out_ref[:c, :] = feat_ref[...]
    out_ref[c:, :] = jnp.transpose(shiftT_ref[...])


def kernel(input, mask):
    b, c, h, w = input.shape
    c2 = c // 2
    L = h * w
    feat = input[0].reshape(c, L)
    maskf = mask.reshape(L)
    latT = jnp.transpose(feat[c2:])
    vT = jnp.transpose(feat[:c2])

    mesh = plsc.VectorSubcoreMesh(core_axis_name="c", subcore_axis_name="s")
    sc1 = pl.kernel(
        _sc_compact_gather,
        mesh=mesh,
        out_type=[
            jax.ShapeDtypeStruct((L, c2), jnp.float32),
            jax.ShapeDtypeStruct((L, c2), jnp.float32),
            jax.ShapeDtypeStruct((L, c2), jnp.float32),
            jax.ShapeDtypeStruct((L,), jnp.int32),
            jax.ShapeDtypeStruct((16,), jnp.int32),
        ],
        scratch_types=[
            pltpu.VMEM((L,), jnp.int32),
            pltpu.VMEM((L + 16,), jnp.int32),
            pltpu.VMEM((L + 16,), jnp.int32),
            pltpu.VMEM((L,), jnp.int32),
            pltpu.VMEM((_CHUNK,), jnp.int32),
            pltpu.VMEM((_CHUNK,), jnp.int32),
            pltpu.VMEM((_CHUNK, c2), jnp.float32),
            pltpu.VMEM((_CHUNK, c2), jnp.float32),
            pltpu.VMEM((_CHUNK, c2), jnp.float32),
            pltpu.VMEM((16,), jnp.int32),
            pltpu.SemaphoreType.DMA,
        ],
        compiler_params=_SC_PARAMS,
    )
    qT, kT, vkT, oidx, counts = sc1(maskf, latT, vT)

    grid_spec = pltpu.PrefetchScalarGridSpec(
        num_scalar_prefetch=1,
        grid=(L // _BM,),
        in_specs=[
            pl.BlockSpec((_BM, c2), lambda i, cnt: (i, 0)),
            pl.BlockSpec((L, c2), lambda i, cnt: (0, 0)),
            pl.BlockSpec((L, c2), lambda i, cnt: (0, 0)),
        ],
        out_specs=pl.BlockSpec((_BM, c2), lambda i, cnt: (i, 0)),
        scratch_shapes=[
            pltpu.VMEM((L, c2), jnp.float32),
            pltpu.VMEM((L, 2 * c2), jnp.bfloat16),
        ],
    )
    o_attn = pl.pallas_call(
        _attn_block,
        grid_spec=grid_spec,
        out_shape=jax.ShapeDtypeStruct((L, c2), jnp.float32),
    )(counts, qT, kT, vkT)

    sc2 = pl.kernel(
        _sc_out_gather,
        mesh=mesh,
        out_type=[jax.ShapeDtypeStruct((L, c2), jnp.float32)],
        scratch_types=[
            pltpu.VMEM((_CHUNK,), jnp.int32),
            pltpu.VMEM((_CHUNK, c2), jnp.float32),
            pltpu.SemaphoreType.DMA,
        ],
        compiler_params=_SC_PARAMS,
    )
    (shiftT,) = sc2(o_attn, oidx)

    out = jnp.concatenate([feat, jnp.transpose(shiftT)], axis=0)
    out = out.reshape(1, c + c2, h, w)
    return jnp.broadcast_to(out, (b, c + c2, h, w))


# speedup vs baseline: 3.3496x; 3.3496x over previous
import jax
import jax.numpy as jnp
from jax.experimental import pallas as pl
from jax.experimental.pallas import tpu as pltpu

_BM = 1024


def _attn_block(feat_ref, known_ref, flag_ref, out_ref, kn_ref, vk_ref, kb_ref):
    c2 = kn_ref.shape[0]
    c = feat_ref.shape[0]
    i = pl.program_id(0)

    @pl.when(i == 0)
    def _prep():
        lat = feat_ref[c2:, :]
        norm = jnp.sqrt(jnp.sum(lat * lat, axis=0, keepdims=True)) + 1e-4
        kn_ref[...] = lat * (1.4426950408889634 / norm)
        vk_ref[...] = (feat_ref[:c2, :] * known_ref[...]).astype(jnp.bfloat16)
        kb_ref[...] = known_ref[...].astype(jnp.bfloat16)

    q = feat_ref[c2:, pl.ds(i * _BM, _BM)]
    s = jax.lax.dot_general(q, kn_ref[...], (((0,), (0,)), ((), ())),
                            preferred_element_type=jnp.float32)
    e = jnp.exp2(s).astype(jnp.bfloat16)
    o = jax.lax.dot_general(vk_ref[...], e, (((1,), (1,)), ((), ())),
                            preferred_element_type=jnp.float32)
    d = jax.lax.dot_general(kb_ref[...], e, (((1,), (1,)), ((), ())),
                            preferred_element_type=jnp.float32)
    out_ref[:c, :] = feat_ref[:, pl.ds(i * _BM, _BM)]
    out_ref[c:, :] = o * (flag_ref[...] / d)


def kernel(input, mask):
    b, c, h, w = input.shape
    c2 = c // 2
    L = h * w
    feat = input[0].reshape(c, L)
    flag = mask.reshape(1, L).astype(jnp.float32)
    known = 1.0 - flag

    grid = (L // _BM,)
    out = pl.pallas_call(
        _attn_block,
        grid=grid,
        in_specs=[
            pl.BlockSpec((c, L), lambda i: (0, 0)),
            pl.BlockSpec((1, L), lambda i: (0, 0)),
            pl.BlockSpec((1, _BM), lambda i: (0, i)),
        ],
        out_specs=pl.BlockSpec((c + c2, _BM), lambda i: (0, i)),
        out_shape=jax.ShapeDtypeStruct((c + c2, L), jnp.float32),
        scratch_shapes=[
            pltpu.VMEM((c2, L), jnp.float32),
            pltpu.VMEM((c2, L), jnp.bfloat16),
            pltpu.VMEM((1, L), jnp.bfloat16),
        ],
    )(feat, known, flag)

    out = out.reshape(1, c + c2, h, w)
    return jnp.broadcast_to(out, (b, c + c2, h, w))


# speedup vs baseline: 3.3827x; 1.0099x over previous
import jax
import jax.numpy as jnp
from jax.experimental import pallas as pl
from jax.experimental.pallas import tpu as pltpu

_BM = 2048


def _attn_block(feat_ref, known_ref, flag_ref, out_ref, kn_ref, vk_ref, kb_ref):
    c2 = kn_ref.shape[0]
    c = feat_ref.shape[0]
    i = pl.program_id(0)

    @pl.when(i == 0)
    def _prep():
        lat = feat_ref[c2:, :]
        norm = jnp.sqrt(jnp.sum(lat * lat, axis=0, keepdims=True)) + 1e-4
        kn_ref[...] = lat * (1.4426950408889634 / norm)
        vk_ref[...] = (feat_ref[:c2, :] * known_ref[...]).astype(jnp.bfloat16)
        kb_ref[...] = known_ref[...].astype(jnp.bfloat16)

    q = feat_ref[c2:, pl.ds(i * _BM, _BM)]
    s = jax.lax.dot_general(q, kn_ref[...], (((0,), (0,)), ((), ())),
                            preferred_element_type=jnp.float32)
    e = jnp.exp2(s).astype(jnp.bfloat16)
    o = jax.lax.dot_general(vk_ref[...], e, (((1,), (1,)), ((), ())),
                            preferred_element_type=jnp.float32)
    d = jax.lax.dot_general(kb_ref[...], e, (((1,), (1,)), ((), ())),
                            preferred_element_type=jnp.float32)
    out_ref[:c, :] = feat_ref[:, pl.ds(i * _BM, _BM)]
    out_ref[c:, :] = o * (flag_ref[...] / d)


def kernel(input, mask):
    b, c, h, w = input.shape
    c2 = c // 2
    L = h * w
    feat = input[0].reshape(c, L)
    flag = mask.reshape(1, L).astype(jnp.float32)
    known = 1.0 - flag

    grid = (L // _BM,)
    out = pl.pallas_call(
        _attn_block,
        grid=grid,
        in_specs=[
            pl.BlockSpec((c, L), lambda i: (0, 0)),
            pl.BlockSpec((1, L), lambda i: (0, 0)),
            pl.BlockSpec((1, _BM), lambda i: (0, i)),
        ],
        out_specs=pl.BlockSpec((c + c2, _BM), lambda i: (0, i)),
        out_shape=jax.ShapeDtypeStruct((c + c2, L), jnp.float32),
        scratch_shapes=[
            pltpu.VMEM((c2, L), jnp.float32),
            pltpu.VMEM((c2, L), jnp.bfloat16),
            pltpu.VMEM((1, L), jnp.bfloat16),
        ],
    )(feat, known, flag)

    out = out.reshape(1, c + c2, h, w)
    return jnp.broadcast_to(out, (b, c + c2, h, w))
